# baseline (speedup 1.0000x reference)
import functools

import jax
import jax.numpy as jnp
from jax import lax
from jax.experimental import pallas as pl
from jax.experimental.pallas import tpu as pltpu, tpu_sc as plsc

_BATCH = 4
_SEQ = 8192
_D = 64

_NC = 2
_NS = 16
_NW = _NC * _NS

_NDC = 8
_DC = _D // _NDC
_NLQ = _NW // _NDC
_LQ = _SEQ // _NLQ
_NV = (_DC * _LQ) // 16


def _pos_add_body(x_hbm, tab_hbm, out_hbm, bufx0, bufx1, buft,
                  semt, semx0, semx1, semo0, semo1):
    wid = lax.axis_index("s") * _NC + lax.axis_index("c")
    dc0 = (wid // _NLQ) * _DC
    l0 = (wid % _NLQ) * _LQ

    bufs = (bufx0, bufx1)
    semx = (semx0, semx1)
    semo = (semo0, semo1)

    ct = pltpu.async_copy(tab_hbm.at[pl.ds(dc0, _DC), pl.ds(l0, _LQ)],
                          buft, semt)
    pltpu.async_copy(x_hbm.at[0, pl.ds(dc0, _DC), pl.ds(l0, _LQ)],
                     bufx0, semx0)
    ct.wait()

    for b in range(_BATCH):
        cur = bufs[b % 2]
        pltpu.make_async_copy(x_hbm.at[b, pl.ds(dc0, _DC), pl.ds(l0, _LQ)],
                              cur, semx[b % 2]).wait()
        if b + 1 < _BATCH:
            nxt = bufs[(b + 1) % 2]
            if b >= 1:
                pltpu.make_async_copy(
                    nxt, out_hbm.at[b - 1, pl.ds(dc0, _DC), pl.ds(l0, _LQ)],
                    semo[(b + 1) % 2]).wait()
            pltpu.async_copy(x_hbm.at[b + 1, pl.ds(dc0, _DC), pl.ds(l0, _LQ)],
                             nxt, semx[(b + 1) % 2])

        @plsc.parallel_loop(0, _NV, unroll=8)
        def _add(i):
            r = i // (_LQ // 16)
            s = pl.ds((i % (_LQ // 16)) * 16, 16)
            plsc.addupdate(cur.at[r, s], buft[r, s])

        pltpu.async_copy(cur, out_hbm.at[b, pl.ds(dc0, _DC), pl.ds(l0, _LQ)],
                         semo[b % 2])

    pltpu.make_async_copy(bufs[(_BATCH - 2) % 2],
                          out_hbm.at[_BATCH - 2, pl.ds(dc0, _DC), pl.ds(l0, _LQ)],
                          semo[(_BATCH - 2) % 2]).wait()
    pltpu.make_async_copy(bufs[(_BATCH - 1) % 2],
                          out_hbm.at[_BATCH - 1, pl.ds(dc0, _DC), pl.ds(l0, _LQ)],
                          semo[(_BATCH - 1) % 2]).wait()


def _make_pos_add(interpret=False):
    return functools.partial(
        pl.kernel,
        out_type=jax.ShapeDtypeStruct((_BATCH, _D, _SEQ), jnp.float32),
        mesh=plsc.VectorSubcoreMesh(core_axis_name="c", subcore_axis_name="s"),
        scratch_types=[
            pltpu.VMEM((_DC, _LQ), jnp.float32),
            pltpu.VMEM((_DC, _LQ), jnp.float32),
            pltpu.VMEM((_DC, _LQ), jnp.float32),
            pltpu.SemaphoreType.DMA,
            pltpu.SemaphoreType.DMA,
            pltpu.SemaphoreType.DMA,
            pltpu.SemaphoreType.DMA,
            pltpu.SemaphoreType.DMA,
        ],
        compiler_params=pltpu.CompilerParams(
            disable_bounds_checks=True,
            disable_semaphore_checks=True,
            skip_device_barrier=True,
        ),
        interpret=interpret,
    )(_pos_add_body)


_pos_add = _make_pos_add()


def kernel(x, pos_emb_table):
    xt = jnp.transpose(x, (0, 2, 1))
    tabt = jnp.transpose(pos_emb_table)
    outt = _pos_add(xt, tabt)
    return jnp.transpose(outt, (0, 2, 1))


# speedup vs baseline: 1.0034x; 1.0034x over previous
import functools

import jax
import jax.numpy as jnp
from jax import lax
from jax.experimental import pallas as pl
from jax.experimental.pallas import tpu as pltpu, tpu_sc as plsc

_BATCH = 4
_SEQ = 8192
_D = 64

_NC = 2
_NS = 16
_NW = _NC * _NS

_NDC = 8
_DC = _D // _NDC
_NLQ = _NW // _NDC
_LQ = _SEQ // _NLQ
_NV = (_DC * _LQ) // 16


def _pos_add_body(x_hbm, tab_hbm, out_hbm, bufa, bufb, bufc, buft,
                  semt, sxa, sxb, sxc, soa, sob, soc):
    wid = lax.axis_index("s") * _NC + lax.axis_index("c")
    dc0 = (wid // _NLQ) * _DC
    l0 = (wid % _NLQ) * _LQ

    def xsl(b):
        return x_hbm.at[b, pl.ds(dc0, _DC), pl.ds(l0, _LQ)]

    def osl(b):
        return out_hbm.at[b, pl.ds(dc0, _DC), pl.ds(l0, _LQ)]

    def add_into(cur):
        @plsc.parallel_loop(0, _NV, unroll=8)
        def _add(i):
            r = i // (_LQ // 16)
            s = pl.ds((i % (_LQ // 16)) * 16, 16)
            plsc.addupdate(cur.at[r, s], buft[r, s])

    ct = pltpu.async_copy(tab_hbm.at[pl.ds(dc0, _DC), pl.ds(l0, _LQ)],
                          buft, semt)
    pltpu.async_copy(xsl(0), bufa, sxa)
    pltpu.async_copy(xsl(1), bufb, sxb)
    ct.wait()

    pltpu.make_async_copy(xsl(0), bufa, sxa).wait()
    pltpu.async_copy(xsl(2), bufc, sxc)
    add_into(bufa)
    pltpu.async_copy(bufa, osl(0), soa)
    pltpu.make_async_copy(xsl(1), bufb, sxb).wait()
    add_into(bufb)
    pltpu.async_copy(bufb, osl(1), sob)
    pltpu.make_async_copy(bufa, osl(0), soa).wait()
    pltpu.async_copy(xsl(3), bufa, sxa)
    pltpu.make_async_copy(xsl(2), bufc, sxc).wait()
    add_into(bufc)
    pltpu.async_copy(bufc, osl(2), soc)
    pltpu.make_async_copy(xsl(3), bufa, sxa).wait()
    add_into(bufa)
    pltpu.async_copy(bufa, osl(3), soa)
    pltpu.make_async_copy(bufb, osl(1), sob).wait()
    pltpu.make_async_copy(bufc, osl(2), soc).wait()
    pltpu.make_async_copy(bufa, osl(3), soa).wait()


def _make_pos_add(interpret=False):
    return functools.partial(
        pl.kernel,
        out_type=jax.ShapeDtypeStruct((_BATCH, _D, _SEQ), jnp.float32),
        mesh=plsc.VectorSubcoreMesh(core_axis_name="c", subcore_axis_name="s"),
        scratch_types=[
            pltpu.VMEM((_DC, _LQ), jnp.float32),
            pltpu.VMEM((_DC, _LQ), jnp.float32),
            pltpu.VMEM((_DC, _LQ), jnp.float32),
            pltpu.VMEM((_DC, _LQ), jnp.float32),
            pltpu.SemaphoreType.DMA,
            pltpu.SemaphoreType.DMA,
            pltpu.SemaphoreType.DMA,
            pltpu.SemaphoreType.DMA,
            pltpu.SemaphoreType.DMA,
            pltpu.SemaphoreType.DMA,
            pltpu.SemaphoreType.DMA,
        ],
        interpret=interpret,
    )(_pos_add_body)


_pos_add = _make_pos_add()


def kernel(x, pos_emb_table):
    xt = jnp.transpose(x, (0, 2, 1))
    tabt = jnp.transpose(pos_emb_table)
    outt = _pos_add(xt, tabt)
    return jnp.transpose(outt, (0, 2, 1))
